# Initial kernel scaffold; baseline (speedup 1.0000x reference)
import functools

import jax
import jax.numpy as jnp
from jax import lax
from jax.experimental import pallas as pl
from jax.experimental.pallas import tpu as pltpu
from jax.experimental.pallas import tpu_sc as plsc

_THRESH = 0.7
_MIN_KEPT = 100000
_DICE_W = 0.5
_BCE_W = 0.5
_SMOOTH = 1.0

_NW = 32
_L = 16
_NSLOT = 2
_B1 = 2048
_B2 = 2048
_B3 = 256
_SH1 = 19
_SH2 = 8
_CHUNK = 8192

_mesh = plsc.VectorSubcoreMesh(core_axis_name="c", subcore_axis_name="s")


def _wid():
    return lax.axis_index("s") * 2 + lax.axis_index("c")


def _ew_body(lg_ref, tg_ref, cb_ref, bce_ref, dn_ref, dd_ref):
    lg = lg_ref[0]
    tg = tg_ref[0].astype(jnp.float32)
    prob = jax.nn.sigmoid(lg)
    conf = jnp.where(tg > 0.5, prob, 1.0 - prob)
    cb_ref[0] = lax.bitcast_convert_type(conf, jnp.int32)
    bce_ref[0] = jnp.maximum(lg, 0.0) - lg * tg + jnp.log1p(jnp.exp(-jnp.abs(lg)))
    dn_ref[...] = jnp.sum(prob * tg, axis=0, keepdims=True)
    dd_ref[...] = jnp.sum(prob + tg, axis=0, keepdims=True)


def _elementwise(lg3, tg3):
    B, R, C = lg3.shape
    blk = lambda b: (b, 0, 0)
    return pl.pallas_call(
        _ew_body,
        grid=(B,),
        in_specs=[pl.BlockSpec((1, R, C), blk), pl.BlockSpec((1, R, C), blk)],
        out_specs=[
            pl.BlockSpec((1, R, C), blk),
            pl.BlockSpec((1, R, C), blk),
            pl.BlockSpec((1, C), lambda b: (b, 0)),
            pl.BlockSpec((1, C), lambda b: (b, 0)),
        ],
        out_shape=[
            jax.ShapeDtypeStruct((B, R, C), jnp.int32),
            jax.ShapeDtypeStruct((B, R, C), jnp.float32),
            jax.ShapeDtypeStruct((B, C), jnp.float32),
            jax.ShapeDtypeStruct((B, C), jnp.float32),
        ],
    )(lg3, tg3)


def _zero_ref(ref, n):
    z = jnp.zeros((_L,), jnp.int32)

    def body(j, _):
        ref[pl.ds(j * _L, _L)] = z
        return 0

    lax.fori_loop(0, n // _L, body, 0)


def _combine(hbm, combuf, hsum, nbins, k):
    for half in range(2):
        pltpu.sync_copy(hbm.at[pl.ds(half * _L, _L)], combuf.at[:, pl.ds(0, nbins)])

        def rbody(j, _):
            acc = combuf[0, pl.ds(j * _L, _L)]
            for r in range(1, _L):
                acc = acc + combuf[r, pl.ds(j * _L, _L)]
            if half == 0:
                hsum[0, pl.ds(j * _L, _L)] = acc
            else:
                hsum[0, pl.ds(j * _L, _L)] = hsum[0, pl.ds(j * _L, _L)] + acc
            return 0

        lax.fori_loop(0, nbins // _L, rbody, 0)

    def sbody(j, car):
        p, csum, carry = car
        h = hsum[0, pl.ds(j * _L, _L)]
        s = plsc.cumsum(h) + carry
        m = s <= k
        p = p + jnp.max(plsc.all_reduce_population_count(m))
        csum = csum + jnp.sum(jnp.where(m, h, 0))
        carry = jnp.max(s)
        return p, csum, carry

    p, csum, _ = lax.fori_loop(
        0, nbins // _L, sbody,
        (jnp.int32(0), jnp.int32(0), jnp.int32(0)))
    return p, k - csum


def _stream_hist(cb_hbm, buf, lhist, sems, n_per_tile, base, nbins, digit_fn):
    nchunks = n_per_tile // _CHUNK
    laneoff = lax.iota(jnp.int32, _L) * nbins
    ones = jnp.ones((_L,), jnp.int32)

    def dma(c):
        return pltpu.make_async_copy(
            cb_hbm.at[pl.ds(base + c * _CHUNK, _CHUNK)], buf.at[c % 2], sems[c % 2])

    dma(0).start()
    for c in range(nchunks):
        s = c % 2
        dma(c).wait()
        if c + 1 < nchunks:
            dma(c + 1).start()

        def ibody(i, _):
            for u in range(_NSLOT):
                v = buf[s, pl.ds(i * (_L * _NSLOT) + u * _L, _L)]
                d, msk = digit_fn(v)
                idx = (u * (_L * nbins) + laneoff) + d
                if msk is None:
                    plsc.addupdate_scatter(lhist, [idx], ones)
                else:
                    plsc.addupdate_scatter(lhist, [idx], ones, mask=msk)
            return 0

        lax.fori_loop(0, _CHUNK // (_L * _NSLOT), ibody, 0)


def _reduce_lanes(lhist, hsum, nbins):
    def body(j, _):
        acc = lhist[pl.ds(j * _L, _L)]
        for r in range(1, _L * _NSLOT):
            acc = acc + lhist[pl.ds(r * nbins + j * _L, _L)]
        hsum[0, pl.ds(j * _L, _L)] = acc
        return 0

    lax.fori_loop(0, nbins // _L, body, 0)


def _make_sc_passes(n_total, k_rank):
    n_per_tile = n_total // _NW

    @functools.partial(
        pl.kernel,
        out_type=jax.ShapeDtypeStruct((_NW, _B1), jnp.int32),
        mesh=_mesh,
        scratch_types=[
            pltpu.VMEM((2, _CHUNK), jnp.int32),
            pltpu.VMEM((_NSLOT * _L * _B1,), jnp.int32),
            pltpu.VMEM((1, _B1), jnp.int32),
            pltpu.SemaphoreType.DMA,
            pltpu.SemaphoreType.DMA,
        ],
    )
    def hist1(cb_hbm, h1_hbm, buf, lhist, hsum, sem0, sem1):
        w = _wid()
        _zero_ref(lhist, _NSLOT * _L * _B1)

        def digit(v):
            return jax.lax.shift_right_logical(v, _SH1), None

        _stream_hist(cb_hbm, buf, lhist, (sem0, sem1),
                     n_per_tile, w * n_per_tile, _B1, digit)
        _reduce_lanes(lhist, hsum, _B1)
        pltpu.sync_copy(hsum, h1_hbm.at[pl.ds(w, 1)])

    @functools.partial(
        pl.kernel,
        out_type=jax.ShapeDtypeStruct((_NW, _B2), jnp.int32),
        mesh=_mesh,
        scratch_types=[
            pltpu.VMEM((2, _CHUNK), jnp.int32),
            pltpu.VMEM((_NSLOT * _L * _B2,), jnp.int32),
            pltpu.VMEM((1, _B2), jnp.int32),
            pltpu.VMEM((_L, _B1), jnp.int32),
            pltpu.SemaphoreType.DMA,
            pltpu.SemaphoreType.DMA,
        ],
    )
    def hist2(cb_hbm, h1_hbm, h2_hbm, buf, lhist, hsum, combuf, sem0, sem1):
        w = _wid()
        p1, _ = _combine(h1_hbm, combuf, hsum, _B1, k_rank)
        _zero_ref(lhist, _NSLOT * _L * _B2)

        def digit(v):
            msk = jax.lax.shift_right_logical(v, _SH1) == p1
            return jnp.bitwise_and(jax.lax.shift_right_logical(v, _SH2), 0x7FF), msk

        _stream_hist(cb_hbm, buf, lhist, (sem0, sem1),
                     n_per_tile, w * n_per_tile, _B2, digit)
        _reduce_lanes(lhist, hsum, _B2)
        pltpu.sync_copy(hsum, h2_hbm.at[pl.ds(w, 1)])

    @functools.partial(
        pl.kernel,
        out_type=jax.ShapeDtypeStruct((_NW, _B3), jnp.int32),
        mesh=_mesh,
        scratch_types=[
            pltpu.VMEM((2, _CHUNK), jnp.int32),
            pltpu.VMEM((_NSLOT * _L * _B3,), jnp.int32),
            pltpu.VMEM((1, _B3), jnp.int32),
            pltpu.VMEM((_L, _B1), jnp.int32),
            pltpu.VMEM((1, _B1), jnp.int32),
            pltpu.SemaphoreType.DMA,
            pltpu.SemaphoreType.DMA,
        ],
    )
    def hist3(cb_hbm, h1_hbm, h2_hbm, h3_hbm, buf, lhist, hsum3, combuf, hsum,
              sem0, sem1):
        w = _wid()
        p1, k1 = _combine(h1_hbm, combuf, hsum, _B1, k_rank)
        p2, _ = _combine(h2_hbm, combuf, hsum, _B2, k1)
        prefix = jnp.bitwise_or(jax.lax.shift_left(p1, 11), p2)
        _zero_ref(lhist, _NSLOT * _L * _B3)

        def digit(v):
            msk = jax.lax.shift_right_logical(v, _SH2) == prefix
            return jnp.bitwise_and(v, 0xFF), msk

        _stream_hist(cb_hbm, buf, lhist, (sem0, sem1),
                     n_per_tile, w * n_per_tile, _B3, digit)
        _reduce_lanes(lhist, hsum3, _B3)
        pltpu.sync_copy(hsum3, h3_hbm.at[pl.ds(w, 1)])

    @functools.partial(
        pl.kernel,
        out_type=jax.ShapeDtypeStruct((_L,), jnp.int32),
        mesh=_mesh,
        scratch_types=[
            pltpu.VMEM((1, _B1), jnp.int32),
            pltpu.VMEM((_L, _B1), jnp.int32),
            pltpu.VMEM((_L,), jnp.int32),
        ],
    )
    def resolve(h1_hbm, h2_hbm, h3_hbm, t_hbm, hsum, combuf, tvm):
        w = _wid()

        @pl.when(w == 0)
        def _():
            p1, k1 = _combine(h1_hbm, combuf, hsum, _B1, k_rank)
            p2, k2 = _combine(h2_hbm, combuf, hsum, _B2, k1)
            p3, _ = _combine(h3_hbm, combuf, hsum, _B3, k2)
            v_bits = jnp.bitwise_or(
                jax.lax.shift_left(p1, _SH1),
                jnp.bitwise_or(jax.lax.shift_left(p2, _SH2), p3))
            vb = jnp.full((_L,), v_bits, jnp.int32)
            t = jnp.maximum(plsc.bitcast(vb, jnp.float32), _THRESH)
            tvm[...] = plsc.bitcast(t, jnp.int32)
            pltpu.sync_copy(tvm, t_hbm)

    return hist1, hist2, hist3, resolve


def _fin_body(tb_ref, cb_ref, bce_ref, dn_ref, dd_ref, out_ref, cnt_sm, bs_sm):
    b = pl.program_id(0)
    m = (cb_ref[0] < tb_ref[0]).astype(jnp.float32)

    @pl.when(b == 0)
    def _():
        cnt_sm[0] = 0.0
        bs_sm[0] = 0.0

    cnt_sm[0] += jnp.sum(m)
    bs_sm[0] += jnp.sum(bce_ref[0] * m)

    @pl.when(b == pl.num_programs(0) - 1)
    def _():
        num = 2.0 * jnp.sum(dn_ref[...], axis=1, keepdims=True) + _SMOOTH
        den = jnp.sum(dd_ref[...], axis=1, keepdims=True) + _SMOOTH
        dice = jnp.mean(1.0 - num / den)
        bce_l = bs_sm[0] / jnp.maximum(cnt_sm[0], 1.0)
        out_ref[0, 0] = _DICE_W * dice + _BCE_W * bce_l


def _finalize(t_bits1, cb3, bce3, dn, dd):
    B, R, C = cb3.shape
    blk = lambda b: (b, 0, 0)
    return pl.pallas_call(
        _fin_body,
        grid=(B,),
        in_specs=[
            pl.BlockSpec(memory_space=pltpu.SMEM),
            pl.BlockSpec((1, R, C), blk),
            pl.BlockSpec((1, R, C), blk),
            pl.BlockSpec((B, C), lambda b: (0, 0)),
            pl.BlockSpec((B, C), lambda b: (0, 0)),
        ],
        out_specs=pl.BlockSpec((1, 1), lambda b: (0, 0)),
        out_shape=jax.ShapeDtypeStruct((1, 1), jnp.float32),
        scratch_shapes=[pltpu.SMEM((1,), jnp.float32), pltpu.SMEM((1,), jnp.float32)],
    )(t_bits1, cb3, bce3, dn, dd)


def kernel(pred_logits, target):
    B, _, H, W = pred_logits.shape
    n = B * H * W
    assert n % (_NW * _CHUNK) == 0
    k_rank = min(_MIN_KEPT * B, n - 1)

    lg3 = pred_logits.reshape(B, (H * W) // 128, 128)
    tg3 = target.reshape(B, (H * W) // 128, 128)

    cb3, bce3, dn, dd = _elementwise(lg3, tg3)

    hist1, hist2, hist3, resolve = _make_sc_passes(n, k_rank)
    cb_flat = cb3.reshape(n)
    h1 = hist1(cb_flat)
    h2 = hist2(cb_flat, h1)
    h3 = hist3(cb_flat, h1, h2)
    t_bits = resolve(h1, h2, h3)

    out = _finalize(t_bits[:1], cb3, bce3, dn, dd)
    return out[0, 0]


# speedup vs baseline: 14.0257x; 14.0257x over previous
import functools

import jax
import jax.numpy as jnp
from jax import lax
from jax.experimental import pallas as pl
from jax.experimental.pallas import tpu as pltpu
from jax.experimental.pallas import tpu_sc as plsc

_THRESH = 0.7
_MIN_KEPT = 100000
_DICE_W = 0.5
_BCE_W = 0.5
_SMOOTH = 1.0

_NW = 32
_L = 16
_NSLOT = 2
_B1 = 2048
_B2 = 2048
_B3 = 256
_SH1 = 19
_SH2 = 8
_CHUNK = 8192

def _get_mesh():
    return plsc.VectorSubcoreMesh(
        core_axis_name="c", subcore_axis_name="s", num_cores=2, num_subcores=16)


def _wid():
    return lax.axis_index("s") * 2 + lax.axis_index("c")


def _ew_body(lg_ref, tg_ref, cb_ref, bce_ref, dn_ref, dd_ref):
    lg = lg_ref[0]
    tg = tg_ref[0].astype(jnp.float32)
    prob = jax.nn.sigmoid(lg)
    conf = jnp.where(tg > 0.5, prob, 1.0 - prob)
    cb_ref[0] = lax.bitcast_convert_type(conf, jnp.int32)
    bce_ref[0] = jnp.maximum(lg, 0.0) - lg * tg + jnp.log1p(jnp.exp(-jnp.abs(lg)))
    dn_ref[0] = jnp.sum(prob * tg, axis=0, keepdims=True)
    dd_ref[0] = jnp.sum(prob + tg, axis=0, keepdims=True)


def _elementwise(lg3, tg3):
    B, R, C = lg3.shape
    blk = lambda b: (b, 0, 0)
    return pl.pallas_call(
        _ew_body,
        grid=(B,),
        in_specs=[pl.BlockSpec((1, R, C), blk), pl.BlockSpec((1, R, C), blk)],
        out_specs=[
            pl.BlockSpec((1, R, C), blk),
            pl.BlockSpec((1, R, C), blk),
            pl.BlockSpec((1, 1, C), lambda b: (b, 0, 0)),
            pl.BlockSpec((1, 1, C), lambda b: (b, 0, 0)),
        ],
        out_shape=[
            jax.ShapeDtypeStruct((B, R, C), jnp.int32),
            jax.ShapeDtypeStruct((B, R, C), jnp.float32),
            jax.ShapeDtypeStruct((B, 1, C), jnp.float32),
            jax.ShapeDtypeStruct((B, 1, C), jnp.float32),
        ],
    )(lg3, tg3)


def _zero_ref(ref, n):
    z = jnp.zeros((_L,), jnp.int32)

    def body(j, _):
        ref[pl.ds(j * _L, _L)] = z
        return 0

    lax.fori_loop(0, n // _L, body, 0)


def _combine(hbm, combuf, hsum, nbins, k):
    for half in range(2):
        pltpu.sync_copy(hbm.at[pl.ds(half * _L, _L)], combuf)

        def rbody(j, _):
            acc = combuf[0, pl.ds(j * _L, _L)]
            for r in range(1, _L):
                acc = acc + combuf[r, pl.ds(j * _L, _L)]
            if half == 0:
                hsum[0, pl.ds(j * _L, _L)] = acc
            else:
                hsum[0, pl.ds(j * _L, _L)] = hsum[0, pl.ds(j * _L, _L)] + acc
            return 0

        lax.fori_loop(0, nbins // _L, rbody, 0)

    def sbody(j, car):
        p, csum, carry = car
        h = hsum[0, pl.ds(j * _L, _L)]
        s = plsc.cumsum(h) + carry
        m = s <= k
        p = p + jnp.max(plsc.all_reduce_population_count(m))
        csum = csum + jnp.sum(jnp.where(m, h, 0))
        carry = jnp.max(s)
        return p, csum, carry

    p, csum, _ = lax.fori_loop(
        0, nbins // _L, sbody,
        (jnp.int32(0), jnp.int32(0), jnp.int32(0)))
    return p, k - csum


def _stream_hist(cb_hbm, buf, lhist, sems, n_per_tile, base, nbins, digit_fn):
    nchunks = n_per_tile // _CHUNK
    laneoff = lax.iota(jnp.int32, _L) * nbins
    ones = jnp.ones((_L,), jnp.int32)

    def dma(c):
        return pltpu.make_async_copy(
            cb_hbm.at[pl.ds(base + c * _CHUNK, _CHUNK)], buf.at[c % 2], sems[c % 2])

    dma(0).start()
    for c in range(nchunks):
        s = c % 2
        dma(c).wait()
        if c + 1 < nchunks:
            dma(c + 1).start()

        def ibody(i, _):
            for u in range(_NSLOT):
                v = buf[s, pl.ds(i * (_L * _NSLOT) + u * _L, _L)]
                d, msk = digit_fn(v)
                idx = (u * (_L * nbins) + laneoff) + d
                if msk is None:
                    plsc.addupdate_scatter(lhist, [idx], ones)
                else:
                    plsc.addupdate_scatter(lhist, [idx], ones, mask=msk)
            return 0

        lax.fori_loop(0, _CHUNK // (_L * _NSLOT), ibody, 0)


def _reduce_lanes(lhist, hsum, nbins):
    def body(j, _):
        acc = lhist[pl.ds(j * _L, _L)]
        for r in range(1, _L * _NSLOT):
            acc = acc + lhist[pl.ds(r * nbins + j * _L, _L)]
        hsum[0, pl.ds(j * _L, _L)] = acc
        return 0

    lax.fori_loop(0, nbins // _L, body, 0)


def _make_sc_passes(n_total, k_rank):
    n_per_tile = n_total // _NW

    @functools.partial(
        pl.kernel,
        out_type=jax.ShapeDtypeStruct((_NW, _B1), jnp.int32),
        mesh=_get_mesh(),
        compiler_params=pltpu.CompilerParams(needs_layout_passes=False),
        scratch_types=[
            pltpu.VMEM((2, _CHUNK), jnp.int32),
            pltpu.VMEM((_NSLOT * _L * _B1,), jnp.int32),
            pltpu.VMEM((1, _B1), jnp.int32),
            pltpu.SemaphoreType.DMA,
            pltpu.SemaphoreType.DMA,
        ],
    )
    def hist1(cb_hbm, h1_hbm, buf, lhist, hsum, sem0, sem1):
        w = _wid()
        _zero_ref(lhist, _NSLOT * _L * _B1)

        def digit(v):
            return jax.lax.shift_right_logical(v, _SH1), None

        _stream_hist(cb_hbm, buf, lhist, (sem0, sem1),
                     n_per_tile, w * n_per_tile, _B1, digit)
        _reduce_lanes(lhist, hsum, _B1)
        pltpu.sync_copy(hsum, h1_hbm.at[pl.ds(w, 1)])

    @functools.partial(
        pl.kernel,
        out_type=jax.ShapeDtypeStruct((_NW, _B2), jnp.int32),
        mesh=_get_mesh(),
        compiler_params=pltpu.CompilerParams(needs_layout_passes=False),
        scratch_types=[
            pltpu.VMEM((2, _CHUNK), jnp.int32),
            pltpu.VMEM((_NSLOT * _L * _B2,), jnp.int32),
            pltpu.VMEM((1, _B2), jnp.int32),
            pltpu.VMEM((_L, _B1), jnp.int32),
            pltpu.SemaphoreType.DMA,
            pltpu.SemaphoreType.DMA,
        ],
    )
    def hist2(cb_hbm, h1_hbm, h2_hbm, buf, lhist, hsum, combuf, sem0, sem1):
        w = _wid()
        p1, _ = _combine(h1_hbm, combuf, hsum, _B1, k_rank)
        _zero_ref(lhist, _NSLOT * _L * _B2)

        def digit(v):
            msk = jax.lax.shift_right_logical(v, _SH1) == p1
            return jnp.bitwise_and(jax.lax.shift_right_logical(v, _SH2), 0x7FF), msk

        _stream_hist(cb_hbm, buf, lhist, (sem0, sem1),
                     n_per_tile, w * n_per_tile, _B2, digit)
        _reduce_lanes(lhist, hsum, _B2)
        pltpu.sync_copy(hsum, h2_hbm.at[pl.ds(w, 1)])

    @functools.partial(
        pl.kernel,
        out_type=jax.ShapeDtypeStruct((_NW, _B3), jnp.int32),
        mesh=_get_mesh(),
        compiler_params=pltpu.CompilerParams(needs_layout_passes=False),
        scratch_types=[
            pltpu.VMEM((2, _CHUNK), jnp.int32),
            pltpu.VMEM((_NSLOT * _L * _B3,), jnp.int32),
            pltpu.VMEM((1, _B3), jnp.int32),
            pltpu.VMEM((_L, _B1), jnp.int32),
            pltpu.VMEM((1, _B1), jnp.int32),
            pltpu.SemaphoreType.DMA,
            pltpu.SemaphoreType.DMA,
        ],
    )
    def hist3(cb_hbm, h1_hbm, h2_hbm, h3_hbm, buf, lhist, hsum3, combuf, hsum,
              sem0, sem1):
        w = _wid()
        p1, k1 = _combine(h1_hbm, combuf, hsum, _B1, k_rank)
        p2, _ = _combine(h2_hbm, combuf, hsum, _B2, k1)
        prefix = jnp.bitwise_or(jax.lax.shift_left(p1, 11), p2)
        _zero_ref(lhist, _NSLOT * _L * _B3)

        def digit(v):
            msk = jax.lax.shift_right_logical(v, _SH2) == prefix
            return jnp.bitwise_and(v, 0xFF), msk

        _stream_hist(cb_hbm, buf, lhist, (sem0, sem1),
                     n_per_tile, w * n_per_tile, _B3, digit)
        _reduce_lanes(lhist, hsum3, _B3)
        pltpu.sync_copy(hsum3, h3_hbm.at[pl.ds(w, 1)])

    @functools.partial(
        pl.kernel,
        out_type=jax.ShapeDtypeStruct((_L,), jnp.int32),
        mesh=_get_mesh(),
        compiler_params=pltpu.CompilerParams(needs_layout_passes=False),
        scratch_types=[
            pltpu.VMEM((1, _B1), jnp.int32),
            pltpu.VMEM((1, _B3), jnp.int32),
            pltpu.VMEM((_L, _B1), jnp.int32),
            pltpu.VMEM((_L, _B3), jnp.int32),
            pltpu.VMEM((_L,), jnp.int32),
        ],
    )
    def resolve(h1_hbm, h2_hbm, h3_hbm, t_hbm, hsum, hsum3, combuf, combuf3, tvm):
        w = _wid()

        @pl.when(w == 0)
        def _():
            p1, k1 = _combine(h1_hbm, combuf, hsum, _B1, k_rank)
            p2, k2 = _combine(h2_hbm, combuf, hsum, _B2, k1)
            p3, _ = _combine(h3_hbm, combuf3, hsum3, _B3, k2)
            v_bits = jnp.bitwise_or(
                jax.lax.shift_left(p1, _SH1),
                jnp.bitwise_or(jax.lax.shift_left(p2, _SH2), p3))
            vb = jnp.full((_L,), v_bits, jnp.int32)
            t = jnp.maximum(plsc.bitcast(vb, jnp.float32), _THRESH)
            tvm[...] = plsc.bitcast(t, jnp.int32)
            pltpu.sync_copy(tvm, t_hbm)

    return hist1, hist2, hist3, resolve


def _fin_body(tb_ref, cb_ref, bce_ref, dn_ref, dd_ref, out_ref, cnt_sm, bs_sm):
    b = pl.program_id(0)
    m = (cb_ref[0] < tb_ref[0]).astype(jnp.float32)

    @pl.when(b == 0)
    def _():
        cnt_sm[0] = 0.0
        bs_sm[0] = 0.0

    cnt_sm[0] += jnp.sum(m)
    bs_sm[0] += jnp.sum(bce_ref[0] * m)

    @pl.when(b == pl.num_programs(0) - 1)
    def _():
        num = 2.0 * jnp.sum(dn_ref[:, 0, :], axis=1, keepdims=True) + _SMOOTH
        den = jnp.sum(dd_ref[:, 0, :], axis=1, keepdims=True) + _SMOOTH
        dice = jnp.mean(1.0 - num / den)
        bce_l = bs_sm[0] / jnp.maximum(cnt_sm[0], 1.0)
        out_ref[0] = _DICE_W * dice + _BCE_W * bce_l


def _finalize(t_bits1, cb3, bce3, dn, dd):
    B, R, C = cb3.shape
    blk = lambda b: (b, 0, 0)
    return pl.pallas_call(
        _fin_body,
        grid=(B,),
        in_specs=[
            pl.BlockSpec(memory_space=pltpu.SMEM),
            pl.BlockSpec((1, R, C), blk),
            pl.BlockSpec((1, R, C), blk),
            pl.BlockSpec((B, 1, C), lambda b: (0, 0, 0)),
            pl.BlockSpec((B, 1, C), lambda b: (0, 0, 0)),
        ],
        out_specs=pl.BlockSpec(memory_space=pltpu.SMEM),
        out_shape=jax.ShapeDtypeStruct((1,), jnp.float32),
        scratch_shapes=[pltpu.SMEM((1,), jnp.float32), pltpu.SMEM((1,), jnp.float32)],
    )(t_bits1, cb3, bce3, dn, dd)


def kernel(pred_logits, target):
    B, _, H, W = pred_logits.shape
    n = B * H * W
    assert n % (_NW * _CHUNK) == 0
    k_rank = min(_MIN_KEPT * B, n - 1)

    lg3 = pred_logits.reshape(B, (H * W) // 128, 128)
    tg3 = target.reshape(B, (H * W) // 128, 128)

    cb3, bce3, dn, dd = _elementwise(lg3, tg3)

    hist1, hist2, hist3, resolve = _make_sc_passes(n, k_rank)
    cb_flat = cb3.reshape(n)
    h1 = hist1(cb_flat)
    h2 = hist2(cb_flat, h1)
    h3 = hist3(cb_flat, h1, h2)
    t_bits = resolve(h1, h2, h3)

    out = _finalize(t_bits[:1], cb3, bce3, dn, dd)
    return out[0]


# speedup vs baseline: 14.6569x; 1.0450x over previous
import functools

import jax
import jax.numpy as jnp
from jax import lax
from jax.experimental import pallas as pl
from jax.experimental.pallas import tpu as pltpu
from jax.experimental.pallas import tpu_sc as plsc

_THRESH = 0.7
_MIN_KEPT = 100000
_DICE_W = 0.5
_BCE_W = 0.5
_SMOOTH = 1.0

_NW = 32
_L = 16
_NSLOT = 2
_B1 = 2048
_B2 = 2048
_B3 = 256
_SH1 = 19
_SH2 = 8
_CHUNK = 8192

def _get_mesh():
    return plsc.VectorSubcoreMesh(
        core_axis_name="c", subcore_axis_name="s", num_cores=2, num_subcores=16)


def _wid():
    return lax.axis_index("s") * 2 + lax.axis_index("c")


def _ew_body(lg_ref, tg_ref, cb_ref, bce_ref, dn_ref, dd_ref):
    lg = lg_ref[0]
    tg = tg_ref[0].astype(jnp.float32)
    prob = jax.nn.sigmoid(lg)
    conf = jnp.where(tg > 0.5, prob, 1.0 - prob)
    cb_ref[0] = lax.bitcast_convert_type(conf, jnp.int32)
    bce_ref[0] = jnp.maximum(lg, 0.0) - lg * tg + jnp.log1p(jnp.exp(-jnp.abs(lg)))
    dn_ref[0] = jnp.sum(prob * tg, axis=0, keepdims=True)
    dd_ref[0] = jnp.sum(prob + tg, axis=0, keepdims=True)


def _elementwise(lg3, tg3):
    B, R, C = lg3.shape
    blk = lambda b: (b, 0, 0)
    return pl.pallas_call(
        _ew_body,
        grid=(B,),
        in_specs=[pl.BlockSpec((1, R, C), blk), pl.BlockSpec((1, R, C), blk)],
        out_specs=[
            pl.BlockSpec((1, R, C), blk),
            pl.BlockSpec((1, R, C), blk),
            pl.BlockSpec((1, 1, C), lambda b: (b, 0, 0)),
            pl.BlockSpec((1, 1, C), lambda b: (b, 0, 0)),
        ],
        out_shape=[
            jax.ShapeDtypeStruct((B, R, C), jnp.int32),
            jax.ShapeDtypeStruct((B, R, C), jnp.float32),
            jax.ShapeDtypeStruct((B, 1, C), jnp.float32),
            jax.ShapeDtypeStruct((B, 1, C), jnp.float32),
        ],
    )(lg3, tg3)


def _hstride(nbins):
    return nbins + 1


def _hwords(nbins):
    n = _NSLOT * _L * _hstride(nbins)
    return ((n + _L - 1) // _L) * _L


def _zero_ref(ref, n):
    z = jnp.zeros((_L,), jnp.int32)

    def body(j, _):
        ref[pl.ds(j * _L, _L)] = z
        return 0

    lax.fori_loop(0, n // _L, body, 0)


def _combine(hbm, combuf, hsum, nbins, k):
    for half in range(2):
        pltpu.sync_copy(hbm.at[pl.ds(half * _L, _L)], combuf)

        def rbody(j, _):
            acc = combuf[0, pl.ds(j * _L, _L)]
            for r in range(1, _L):
                acc = acc + combuf[r, pl.ds(j * _L, _L)]
            if half == 0:
                hsum[0, pl.ds(j * _L, _L)] = acc
            else:
                hsum[0, pl.ds(j * _L, _L)] = hsum[0, pl.ds(j * _L, _L)] + acc
            return 0

        lax.fori_loop(0, nbins // _L, rbody, 0)

    def sbody(j, car):
        p, csum, carry = car
        h = hsum[0, pl.ds(j * _L, _L)]
        s = plsc.cumsum(h) + carry
        m = s <= k
        p = p + jnp.max(plsc.all_reduce_population_count(m))
        csum = csum + jnp.sum(jnp.where(m, h, 0))
        carry = jnp.max(s)
        return p, csum, carry

    p, csum, _ = lax.fori_loop(
        0, nbins // _L, sbody,
        (jnp.int32(0), jnp.int32(0), jnp.int32(0)))
    return p, k - csum


def _stream_hist(cb_hbm, buf, lhist, sems, n_per_tile, base, nbins, digit_fn):
    nchunks = n_per_tile // _CHUNK
    stride = _hstride(nbins)
    laneoff = lax.iota(jnp.int32, _L) * stride
    ones = jnp.ones((_L,), jnp.int32)
    unroll = 2
    step = _L * _NSLOT * unroll

    def dma(c):
        return pltpu.make_async_copy(
            cb_hbm.at[pl.ds(base + c * _CHUNK, _CHUNK)], buf.at[c % 2], sems[c % 2])

    dma(0).start()
    for c in range(nchunks):
        s = c % 2
        dma(c).wait()
        if c + 1 < nchunks:
            dma(c + 1).start()

        def ibody(i, _):
            for h in range(unroll):
                for u in range(_NSLOT):
                    v = buf[s, pl.ds(i * step + (h * _NSLOT + u) * _L, _L)]
                    d, msk = digit_fn(v)
                    idx = (u * (_L * stride) + laneoff) + d
                    if msk is None:
                        plsc.addupdate_scatter(lhist, [idx], ones)
                    else:
                        plsc.addupdate_scatter(lhist, [idx], ones, mask=msk)
            return 0

        lax.fori_loop(0, _CHUNK // step, ibody, 0)


def _reduce_lanes(lhist, hsum, nbins):
    stride = _hstride(nbins)

    def body(j, _):
        acc = lhist[pl.ds(j * _L, _L)]
        for r in range(1, _L * _NSLOT):
            acc = acc + lhist[pl.ds(r * stride + j * _L, _L)]
        hsum[0, pl.ds(j * _L, _L)] = acc
        return 0

    lax.fori_loop(0, nbins // _L, body, 0)


def _make_sc_passes(n_total, k_rank):
    n_per_tile = n_total // _NW

    @functools.partial(
        pl.kernel,
        out_type=jax.ShapeDtypeStruct((_NW, _B1), jnp.int32),
        mesh=_get_mesh(),
        compiler_params=pltpu.CompilerParams(needs_layout_passes=False),
        scratch_types=[
            pltpu.VMEM((2, _CHUNK), jnp.int32),
            pltpu.VMEM((_hwords(_B1),), jnp.int32),
            pltpu.VMEM((1, _B1), jnp.int32),
            pltpu.SemaphoreType.DMA,
            pltpu.SemaphoreType.DMA,
        ],
    )
    def hist1(cb_hbm, h1_hbm, buf, lhist, hsum, sem0, sem1):
        w = _wid()
        _zero_ref(lhist, _hwords(_B1))

        def digit(v):
            return jax.lax.shift_right_logical(v, _SH1), None

        _stream_hist(cb_hbm, buf, lhist, (sem0, sem1),
                     n_per_tile, w * n_per_tile, _B1, digit)
        _reduce_lanes(lhist, hsum, _B1)
        pltpu.sync_copy(hsum, h1_hbm.at[pl.ds(w, 1)])

    @functools.partial(
        pl.kernel,
        out_type=jax.ShapeDtypeStruct((_NW, _B2), jnp.int32),
        mesh=_get_mesh(),
        compiler_params=pltpu.CompilerParams(needs_layout_passes=False),
        scratch_types=[
            pltpu.VMEM((2, _CHUNK), jnp.int32),
            pltpu.VMEM((_hwords(_B2),), jnp.int32),
            pltpu.VMEM((1, _B2), jnp.int32),
            pltpu.VMEM((_L, _B1), jnp.int32),
            pltpu.SemaphoreType.DMA,
            pltpu.SemaphoreType.DMA,
        ],
    )
    def hist2(cb_hbm, h1_hbm, h2_hbm, buf, lhist, hsum, combuf, sem0, sem1):
        w = _wid()
        p1, _ = _combine(h1_hbm, combuf, hsum, _B1, k_rank)
        _zero_ref(lhist, _hwords(_B2))

        def digit(v):
            msk = jax.lax.shift_right_logical(v, _SH1) == p1
            return jnp.bitwise_and(jax.lax.shift_right_logical(v, _SH2), 0x7FF), msk

        _stream_hist(cb_hbm, buf, lhist, (sem0, sem1),
                     n_per_tile, w * n_per_tile, _B2, digit)
        _reduce_lanes(lhist, hsum, _B2)
        pltpu.sync_copy(hsum, h2_hbm.at[pl.ds(w, 1)])

    @functools.partial(
        pl.kernel,
        out_type=jax.ShapeDtypeStruct((_NW, _B3), jnp.int32),
        mesh=_get_mesh(),
        compiler_params=pltpu.CompilerParams(needs_layout_passes=False),
        scratch_types=[
            pltpu.VMEM((2, _CHUNK), jnp.int32),
            pltpu.VMEM((_hwords(_B3),), jnp.int32),
            pltpu.VMEM((1, _B3), jnp.int32),
            pltpu.VMEM((_L, _B1), jnp.int32),
            pltpu.VMEM((1, _B1), jnp.int32),
            pltpu.SemaphoreType.DMA,
            pltpu.SemaphoreType.DMA,
        ],
    )
    def hist3(cb_hbm, h1_hbm, h2_hbm, h3_hbm, buf, lhist, hsum3, combuf, hsum,
              sem0, sem1):
        w = _wid()
        p1, k1 = _combine(h1_hbm, combuf, hsum, _B1, k_rank)
        p2, _ = _combine(h2_hbm, combuf, hsum, _B2, k1)
        prefix = jnp.bitwise_or(jax.lax.shift_left(p1, 11), p2)
        _zero_ref(lhist, _hwords(_B3))

        def digit(v):
            msk = jax.lax.shift_right_logical(v, _SH2) == prefix
            return jnp.bitwise_and(v, 0xFF), msk

        _stream_hist(cb_hbm, buf, lhist, (sem0, sem1),
                     n_per_tile, w * n_per_tile, _B3, digit)
        _reduce_lanes(lhist, hsum3, _B3)
        pltpu.sync_copy(hsum3, h3_hbm.at[pl.ds(w, 1)])

    @functools.partial(
        pl.kernel,
        out_type=jax.ShapeDtypeStruct((_L,), jnp.int32),
        mesh=_get_mesh(),
        compiler_params=pltpu.CompilerParams(needs_layout_passes=False),
        scratch_types=[
            pltpu.VMEM((1, _B1), jnp.int32),
            pltpu.VMEM((1, _B3), jnp.int32),
            pltpu.VMEM((_L, _B1), jnp.int32),
            pltpu.VMEM((_L, _B3), jnp.int32),
            pltpu.VMEM((_L,), jnp.int32),
        ],
    )
    def resolve(h1_hbm, h2_hbm, h3_hbm, t_hbm, hsum, hsum3, combuf, combuf3, tvm):
        w = _wid()

        @pl.when(w == 0)
        def _():
            p1, k1 = _combine(h1_hbm, combuf, hsum, _B1, k_rank)
            p2, k2 = _combine(h2_hbm, combuf, hsum, _B2, k1)
            p3, _ = _combine(h3_hbm, combuf3, hsum3, _B3, k2)
            v_bits = jnp.bitwise_or(
                jax.lax.shift_left(p1, _SH1),
                jnp.bitwise_or(jax.lax.shift_left(p2, _SH2), p3))
            vb = jnp.full((_L,), v_bits, jnp.int32)
            t = jnp.maximum(plsc.bitcast(vb, jnp.float32), _THRESH)
            tvm[...] = plsc.bitcast(t, jnp.int32)
            pltpu.sync_copy(tvm, t_hbm)

    return hist1, hist2, hist3, resolve


def _fin_body(tb_ref, cb_ref, bce_ref, dn_ref, dd_ref, out_ref, cnt_sm, bs_sm):
    b = pl.program_id(0)
    m = (cb_ref[0] < tb_ref[0]).astype(jnp.float32)

    @pl.when(b == 0)
    def _():
        cnt_sm[0] = 0.0
        bs_sm[0] = 0.0

    cnt_sm[0] += jnp.sum(m)
    bs_sm[0] += jnp.sum(bce_ref[0] * m)

    @pl.when(b == pl.num_programs(0) - 1)
    def _():
        num = 2.0 * jnp.sum(dn_ref[:, 0, :], axis=1, keepdims=True) + _SMOOTH
        den = jnp.sum(dd_ref[:, 0, :], axis=1, keepdims=True) + _SMOOTH
        dice = jnp.mean(1.0 - num / den)
        bce_l = bs_sm[0] / jnp.maximum(cnt_sm[0], 1.0)
        out_ref[0] = _DICE_W * dice + _BCE_W * bce_l


def _finalize(t_bits1, cb3, bce3, dn, dd):
    B, R, C = cb3.shape
    blk = lambda b: (b, 0, 0)
    return pl.pallas_call(
        _fin_body,
        grid=(B,),
        in_specs=[
            pl.BlockSpec(memory_space=pltpu.SMEM),
            pl.BlockSpec((1, R, C), blk),
            pl.BlockSpec((1, R, C), blk),
            pl.BlockSpec((B, 1, C), lambda b: (0, 0, 0)),
            pl.BlockSpec((B, 1, C), lambda b: (0, 0, 0)),
        ],
        out_specs=pl.BlockSpec(memory_space=pltpu.SMEM),
        out_shape=jax.ShapeDtypeStruct((1,), jnp.float32),
        scratch_shapes=[pltpu.SMEM((1,), jnp.float32), pltpu.SMEM((1,), jnp.float32)],
    )(t_bits1, cb3, bce3, dn, dd)


def kernel(pred_logits, target):
    B, _, H, W = pred_logits.shape
    n = B * H * W
    assert n % (_NW * _CHUNK) == 0
    k_rank = min(_MIN_KEPT * B, n - 1)

    lg3 = pred_logits.reshape(B, (H * W) // 128, 128)
    tg3 = target.reshape(B, (H * W) // 128, 128)

    cb3, bce3, dn, dd = _elementwise(lg3, tg3)

    hist1, hist2, hist3, resolve = _make_sc_passes(n, k_rank)
    cb_flat = cb3.reshape(n)
    h1 = hist1(cb_flat)
    h2 = hist2(cb_flat, h1)
    h3 = hist3(cb_flat, h1, h2)
    t_bits = resolve(h1, h2, h3)

    out = _finalize(t_bits[:1], cb3, bce3, dn, dd)
    return out[0]


# speedup vs baseline: 15.7678x; 1.0758x over previous
import functools

import jax
import jax.numpy as jnp
from jax import lax
from jax.experimental import pallas as pl
from jax.experimental.pallas import tpu as pltpu
from jax.experimental.pallas import tpu_sc as plsc

_THRESH = 0.7
_MIN_KEPT = 100000
_DICE_W = 0.5
_BCE_W = 0.5
_SMOOTH = 1.0

_NW = 32
_L = 16
_NSLOT = 2
_B1 = 2048
_B2 = 2048
_B3 = 256
_SH1 = 19
_SH2 = 8
_CHUNK = 8192

def _get_mesh():
    return plsc.VectorSubcoreMesh(
        core_axis_name="c", subcore_axis_name="s", num_cores=2, num_subcores=16)


def _wid():
    return lax.axis_index("s") * 2 + lax.axis_index("c")


def _ew_body(lg_ref, tg_ref, cb_ref, bce_ref, dn_ref, dd_ref):
    lg = lg_ref[0]
    tg = tg_ref[0].astype(jnp.float32)
    prob = jax.nn.sigmoid(lg)
    conf = jnp.where(tg > 0.5, prob, 1.0 - prob)
    cb_ref[0] = lax.bitcast_convert_type(conf, jnp.int32)
    bce_ref[0] = jnp.maximum(lg, 0.0) - lg * tg + jnp.log1p(jnp.exp(-jnp.abs(lg)))
    dn_ref[0] = jnp.sum(prob * tg, axis=0, keepdims=True)
    dd_ref[0] = jnp.sum(prob + tg, axis=0, keepdims=True)


def _elementwise(lg3, tg3):
    B, R, C = lg3.shape
    blk = lambda b: (b, 0, 0)
    return pl.pallas_call(
        _ew_body,
        grid=(B,),
        in_specs=[pl.BlockSpec((1, R, C), blk), pl.BlockSpec((1, R, C), blk)],
        out_specs=[
            pl.BlockSpec((1, R, C), blk),
            pl.BlockSpec((1, R, C), blk),
            pl.BlockSpec((1, 1, C), lambda b: (b, 0, 0)),
            pl.BlockSpec((1, 1, C), lambda b: (b, 0, 0)),
        ],
        out_shape=[
            jax.ShapeDtypeStruct((B, R, C), jnp.int32),
            jax.ShapeDtypeStruct((B, R, C), jnp.float32),
            jax.ShapeDtypeStruct((B, 1, C), jnp.float32),
            jax.ShapeDtypeStruct((B, 1, C), jnp.float32),
        ],
    )(lg3, tg3)


def _hstride(nbins):
    return nbins + 1


def _hwords(nbins):
    n = _NSLOT * _L * _hstride(nbins)
    return ((n + _L - 1) // _L) * _L


def _zero_ref(ref, n):
    z = jnp.zeros((_L,), jnp.int32)

    def body(j, _):
        ref[pl.ds(j * _L, _L)] = z
        return 0

    lax.fori_loop(0, n // _L, body, 0)


def _combine(hbm, combuf, hsum, nbins, k):
    for half in range(2):
        pltpu.sync_copy(hbm.at[pl.ds(half * _L, _L)], combuf)

        def rbody(j, _):
            acc = combuf[0, pl.ds(j * _L, _L)]
            for r in range(1, _L):
                acc = acc + combuf[r, pl.ds(j * _L, _L)]
            if half == 0:
                hsum[0, pl.ds(j * _L, _L)] = acc
            else:
                hsum[0, pl.ds(j * _L, _L)] = hsum[0, pl.ds(j * _L, _L)] + acc
            return 0

        lax.fori_loop(0, nbins // _L, rbody, 0)

    def sbody(j, car):
        p, csum, carry = car
        h = hsum[0, pl.ds(j * _L, _L)]
        s = plsc.cumsum(h) + carry
        m = s <= k
        p = p + jnp.max(plsc.all_reduce_population_count(m))
        csum = csum + jnp.sum(jnp.where(m, h, 0))
        carry = jnp.max(s)
        return p, csum, carry

    p, csum, _ = lax.fori_loop(
        0, nbins // _L, sbody,
        (jnp.int32(0), jnp.int32(0), jnp.int32(0)))
    return p, k - csum


def _stream_hist(cb_hbm, buf, lhist, sems, n_per_tile, base, nbins, digit_fn):
    nchunks = n_per_tile // _CHUNK
    stride = _hstride(nbins)
    laneoff = lax.iota(jnp.int32, _L) * stride
    ones = jnp.ones((_L,), jnp.int32)
    unroll = 4
    step = _L * _NSLOT * unroll

    def dma(c):
        return pltpu.make_async_copy(
            cb_hbm.at[pl.ds(base + c * _CHUNK, _CHUNK)], buf.at[c % 2], sems[c % 2])

    dma(0).start()
    for c in range(nchunks):
        s = c % 2
        dma(c).wait()
        if c + 1 < nchunks:
            dma(c + 1).start()

        def ibody(i, _):
            for h in range(unroll):
                for u in range(_NSLOT):
                    v = buf[s, pl.ds(i * step + (h * _NSLOT + u) * _L, _L)]
                    d, msk = digit_fn(v)
                    idx = (u * (_L * stride) + laneoff) + d
                    if msk is None:
                        plsc.addupdate_scatter(lhist, [idx], ones)
                    else:
                        plsc.addupdate_scatter(lhist, [idx], ones, mask=msk)
            return 0

        lax.fori_loop(0, _CHUNK // step, ibody, 0)


def _reduce_lanes(lhist, hsum, nbins):
    stride = _hstride(nbins)

    def body(j, _):
        acc = lhist[pl.ds(j * _L, _L)]
        for r in range(1, _L * _NSLOT):
            acc = acc + lhist[pl.ds(r * stride + j * _L, _L)]
        hsum[0, pl.ds(j * _L, _L)] = acc
        return 0

    lax.fori_loop(0, nbins // _L, body, 0)


def _make_sc_passes(n_total, k_rank):
    n_per_tile = n_total // _NW

    @functools.partial(
        pl.kernel,
        out_type=jax.ShapeDtypeStruct((_NW, _B1), jnp.int32),
        mesh=_get_mesh(),
        compiler_params=pltpu.CompilerParams(needs_layout_passes=False),
        scratch_types=[
            pltpu.VMEM((2, _CHUNK), jnp.int32),
            pltpu.VMEM((_hwords(_B1),), jnp.int32),
            pltpu.VMEM((1, _B1), jnp.int32),
            pltpu.SemaphoreType.DMA,
            pltpu.SemaphoreType.DMA,
        ],
    )
    def hist1(cb_hbm, h1_hbm, buf, lhist, hsum, sem0, sem1):
        w = _wid()
        _zero_ref(lhist, _hwords(_B1))

        def digit(v):
            return jax.lax.shift_right_logical(v, _SH1), None

        _stream_hist(cb_hbm, buf, lhist, (sem0, sem1),
                     n_per_tile, w * n_per_tile, _B1, digit)
        _reduce_lanes(lhist, hsum, _B1)
        pltpu.sync_copy(hsum, h1_hbm.at[pl.ds(w, 1)])

    @functools.partial(
        pl.kernel,
        out_type=jax.ShapeDtypeStruct((_NW, _B2), jnp.int32),
        mesh=_get_mesh(),
        compiler_params=pltpu.CompilerParams(needs_layout_passes=False),
        scratch_types=[
            pltpu.VMEM((2, _CHUNK), jnp.int32),
            pltpu.VMEM((_hwords(_B2),), jnp.int32),
            pltpu.VMEM((1, _B2), jnp.int32),
            pltpu.VMEM((_L, _B1), jnp.int32),
            pltpu.SemaphoreType.DMA,
            pltpu.SemaphoreType.DMA,
        ],
    )
    def hist2(cb_hbm, h1_hbm, h2_hbm, buf, lhist, hsum, combuf, sem0, sem1):
        w = _wid()
        p1, _ = _combine(h1_hbm, combuf, hsum, _B1, k_rank)
        _zero_ref(lhist, _hwords(_B2))

        def digit(v):
            msk = jax.lax.shift_right_logical(v, _SH1) == p1
            return jnp.bitwise_and(jax.lax.shift_right_logical(v, _SH2), 0x7FF), msk

        _stream_hist(cb_hbm, buf, lhist, (sem0, sem1),
                     n_per_tile, w * n_per_tile, _B2, digit)
        _reduce_lanes(lhist, hsum, _B2)
        pltpu.sync_copy(hsum, h2_hbm.at[pl.ds(w, 1)])

    @functools.partial(
        pl.kernel,
        out_type=jax.ShapeDtypeStruct((_NW, _B3), jnp.int32),
        mesh=_get_mesh(),
        compiler_params=pltpu.CompilerParams(needs_layout_passes=False),
        scratch_types=[
            pltpu.VMEM((2, _CHUNK), jnp.int32),
            pltpu.VMEM((_hwords(_B3),), jnp.int32),
            pltpu.VMEM((1, _B3), jnp.int32),
            pltpu.VMEM((_L, _B1), jnp.int32),
            pltpu.VMEM((1, _B1), jnp.int32),
            pltpu.SemaphoreType.DMA,
            pltpu.SemaphoreType.DMA,
        ],
    )
    def hist3(cb_hbm, h1_hbm, h2_hbm, h3_hbm, buf, lhist, hsum3, combuf, hsum,
              sem0, sem1):
        w = _wid()
        p1, k1 = _combine(h1_hbm, combuf, hsum, _B1, k_rank)
        p2, _ = _combine(h2_hbm, combuf, hsum, _B2, k1)
        prefix = jnp.bitwise_or(jax.lax.shift_left(p1, 11), p2)
        _zero_ref(lhist, _hwords(_B3))

        def digit(v):
            msk = jax.lax.shift_right_logical(v, _SH2) == prefix
            return jnp.bitwise_and(v, 0xFF), msk

        _stream_hist(cb_hbm, buf, lhist, (sem0, sem1),
                     n_per_tile, w * n_per_tile, _B3, digit)
        _reduce_lanes(lhist, hsum3, _B3)
        pltpu.sync_copy(hsum3, h3_hbm.at[pl.ds(w, 1)])

    return hist1, hist2, hist3


def _scan_level(h_ref, rows, k, tri128, off_tri):
    hs = jnp.sum(h_ref[...].astype(jnp.float32), axis=0).reshape(rows, 128)
    cw = jax.lax.dot_general(hs, tri128, (((1,), (0,)), ((), ())),
                             preferred_element_type=jnp.float32)
    tot = cw[:, 127:128].reshape(1, rows)
    off = jax.lax.dot_general(tot, off_tri[:rows, :rows],
                              (((1,), (0,)), ((), ())),
                              preferred_element_type=jnp.float32)
    cum = cw + off.reshape(rows, 1)
    kf = k.astype(jnp.float32)
    sel = cum <= kf
    p = jnp.sum(sel.astype(jnp.int32))
    csum = jnp.max(jnp.where(sel, cum, 0.0))
    return p, k - csum.astype(jnp.int32)


def _fin_body(k_rank, h1_ref, h2_ref, h3_ref, cb_ref, bce_ref, dn_ref, dd_ref,
              out_ref, cnt_sm, bs_sm, tb_sm):
    b = pl.program_id(0)

    @pl.when(b == 0)
    def _():
        rI = lax.broadcasted_iota(jnp.int32, (128, 128), 0)
        cI = lax.broadcasted_iota(jnp.int32, (128, 128), 1)
        tri128 = (rI <= cI).astype(jnp.float32)
        off_tri = (rI < cI).astype(jnp.float32)
        k = jnp.int32(k_rank)
        p1, k1 = _scan_level(h1_ref, _B1 // 128, k, tri128, off_tri)
        p2, k2 = _scan_level(h2_ref, _B2 // 128, k1, tri128, off_tri)
        p3, _ = _scan_level(h3_ref, _B3 // 128, k2, tri128, off_tri)
        vb = jnp.bitwise_or(
            lax.shift_left(p1, _SH1),
            jnp.bitwise_or(lax.shift_left(p2, _SH2), p3))
        tf = lax.bitcast_convert_type(jnp.full((1, 1), vb, jnp.int32),
                                      jnp.float32)
        tb = lax.bitcast_convert_type(jnp.maximum(tf, _THRESH), jnp.int32)
        tb_sm[0] = tb[0, 0]
        cnt_sm[0] = 0.0
        bs_sm[0] = 0.0

    m = (cb_ref[0] < tb_sm[0]).astype(jnp.float32)
    cnt_sm[0] += jnp.sum(m)
    bs_sm[0] += jnp.sum(bce_ref[0] * m)

    @pl.when(b == pl.num_programs(0) - 1)
    def _():
        num = 2.0 * jnp.sum(dn_ref[:, 0, :], axis=1, keepdims=True) + _SMOOTH
        den = jnp.sum(dd_ref[:, 0, :], axis=1, keepdims=True) + _SMOOTH
        dice = jnp.mean(1.0 - num / den)
        bce_l = bs_sm[0] / jnp.maximum(cnt_sm[0], 1.0)
        out_ref[0] = _DICE_W * dice + _BCE_W * bce_l


def _finalize(k_rank, h1, h2, h3, cb3, bce3, dn, dd):
    B, R, C = cb3.shape
    blk = lambda b: (b, 0, 0)
    full2 = lambda b: (0, 0)
    return pl.pallas_call(
        functools.partial(_fin_body, k_rank),
        grid=(B,),
        in_specs=[
            pl.BlockSpec((_NW, _B1), full2),
            pl.BlockSpec((_NW, _B2), full2),
            pl.BlockSpec((_NW, _B3), full2),
            pl.BlockSpec((1, R, C), blk),
            pl.BlockSpec((1, R, C), blk),
            pl.BlockSpec((B, 1, C), lambda b: (0, 0, 0)),
            pl.BlockSpec((B, 1, C), lambda b: (0, 0, 0)),
        ],
        out_specs=pl.BlockSpec(memory_space=pltpu.SMEM),
        out_shape=jax.ShapeDtypeStruct((1,), jnp.float32),
        scratch_shapes=[pltpu.SMEM((1,), jnp.float32),
                        pltpu.SMEM((1,), jnp.float32),
                        pltpu.SMEM((1,), jnp.int32)],
    )(h1, h2, h3, cb3, bce3, dn, dd)


def kernel(pred_logits, target):
    B, _, H, W = pred_logits.shape
    n = B * H * W
    assert n % (_NW * _CHUNK) == 0
    k_rank = min(_MIN_KEPT * B, n - 1)

    lg3 = pred_logits.reshape(B, (H * W) // 128, 128)
    tg3 = target.reshape(B, (H * W) // 128, 128)

    cb3, bce3, dn, dd = _elementwise(lg3, tg3)

    hist1, hist2, hist3 = _make_sc_passes(n, k_rank)
    cb_flat = cb3.reshape(n)
    h1 = hist1(cb_flat)
    h2 = hist2(cb_flat, h1)
    h3 = hist3(cb_flat, h1, h2)

    out = _finalize(k_rank, h1, h2, h3, cb3, bce3, dn, dd)
    return out[0]


# speedup vs baseline: 17.1930x; 1.0904x over previous
import functools

import jax
import jax.numpy as jnp
from jax import lax
from jax.experimental import pallas as pl
from jax.experimental.pallas import tpu as pltpu
from jax.experimental.pallas import tpu_sc as plsc

_THRESH = 0.7
_MIN_KEPT = 100000
_DICE_W = 0.5
_BCE_W = 0.5
_SMOOTH = 1.0

_NW = 32
_L = 16
_NSLOT = 2
_B1 = 2048
_B2 = 2048
_B3 = 256
_SH1 = 19
_SH2 = 8
_CHUNK = 8192

def _get_mesh():
    return plsc.VectorSubcoreMesh(
        core_axis_name="c", subcore_axis_name="s", num_cores=2, num_subcores=16)


def _wid():
    return lax.axis_index("s") * 2 + lax.axis_index("c")


def _ew_body(lg_ref, tg_ref, cb_ref, dn_ref, dd_ref):
    lg = lg_ref[0]
    tg = tg_ref[0].astype(jnp.float32)
    prob = jax.nn.sigmoid(lg)
    conf = jnp.where(tg > 0.5, prob, 1.0 - prob)
    cb_ref[0] = lax.bitcast_convert_type(conf, jnp.int32)
    dn_ref[0] = jnp.sum(prob * tg, axis=0, keepdims=True)
    dd_ref[0] = jnp.sum(prob + tg, axis=0, keepdims=True)


def _elementwise(lg3, tg3):
    B, R, C = lg3.shape
    blk = lambda b: (b, 0, 0)
    return pl.pallas_call(
        _ew_body,
        grid=(B,),
        in_specs=[pl.BlockSpec((1, R, C), blk), pl.BlockSpec((1, R, C), blk)],
        out_specs=[
            pl.BlockSpec((1, R, C), blk),
            pl.BlockSpec((1, 1, C), lambda b: (b, 0, 0)),
            pl.BlockSpec((1, 1, C), lambda b: (b, 0, 0)),
        ],
        out_shape=[
            jax.ShapeDtypeStruct((B, R, C), jnp.int32),
            jax.ShapeDtypeStruct((B, 1, C), jnp.float32),
            jax.ShapeDtypeStruct((B, 1, C), jnp.float32),
        ],
    )(lg3, tg3)


def _hstride(nbins):
    return nbins + 1


def _hwords(nbins):
    n = _NSLOT * _L * _hstride(nbins)
    blk = _L * 8
    return ((n + blk - 1) // blk) * blk


def _zero_ref(ref, n):
    z = jnp.zeros((_L,), jnp.int32)

    def body(j, _):
        for u in range(8):
            ref[pl.ds(j * (_L * 8) + u * _L, _L)] = z
        return 0

    lax.fori_loop(0, n // (_L * 8), body, 0)


def _combine(hbm, combuf, hsum, nbins, k):
    for half in range(2):
        pltpu.sync_copy(hbm.at[pl.ds(half * _L, _L)], combuf)

        def rbody(j, _):
            acc = combuf[0, pl.ds(j * _L, _L)]
            for r in range(1, _L):
                acc = acc + combuf[r, pl.ds(j * _L, _L)]
            if half == 0:
                hsum[0, pl.ds(j * _L, _L)] = acc
            else:
                hsum[0, pl.ds(j * _L, _L)] = hsum[0, pl.ds(j * _L, _L)] + acc
            return 0

        lax.fori_loop(0, nbins // _L, rbody, 0)

    def sbody(j, car):
        p, csum, carry = car
        h = hsum[0, pl.ds(j * _L, _L)]
        s = plsc.cumsum(h) + carry
        m = s <= k
        p = p + jnp.max(plsc.all_reduce_population_count(m))
        csum = csum + jnp.sum(jnp.where(m, h, 0))
        carry = jnp.max(s)
        return p, csum, carry

    p, csum, _ = lax.fori_loop(
        0, nbins // _L, sbody,
        (jnp.int32(0), jnp.int32(0), jnp.int32(0)))
    return p, k - csum


def _stream_hist(cb_hbm, buf, lhist, sems, n_per_tile, base, nbins, digit_fn):
    nchunks = n_per_tile // _CHUNK
    stride = _hstride(nbins)
    laneoff = lax.iota(jnp.int32, _L) * stride
    ones = jnp.ones((_L,), jnp.int32)
    unroll = 4
    step = _L * _NSLOT * unroll

    def dma(c):
        return pltpu.make_async_copy(
            cb_hbm.at[pl.ds(base + c * _CHUNK, _CHUNK)], buf.at[c % 2], sems[c % 2])

    dma(0).start()
    for c in range(nchunks):
        s = c % 2
        dma(c).wait()
        if c + 1 < nchunks:
            dma(c + 1).start()

        def ibody(i, _):
            for h in range(unroll):
                for u in range(_NSLOT):
                    v = buf[s, pl.ds(i * step + (h * _NSLOT + u) * _L, _L)]
                    d, msk = digit_fn(v)
                    idx = (u * (_L * stride) + laneoff) + d
                    if msk is None:
                        plsc.addupdate_scatter(lhist, [idx], ones)
                    else:
                        plsc.addupdate_scatter(lhist, [idx], ones, mask=msk)
            return 0

        lax.fori_loop(0, _CHUNK // step, ibody, 0)


def _reduce_lanes(lhist, hsum, nbins):
    stride = _hstride(nbins)

    def body(j, _):
        acc = lhist[pl.ds(j * _L, _L)]
        for r in range(1, _L * _NSLOT):
            acc = acc + lhist[pl.ds(r * stride + j * _L, _L)]
        hsum[0, pl.ds(j * _L, _L)] = acc
        return 0

    lax.fori_loop(0, nbins // _L, body, 0)


def _make_sc_passes(n_total, k_rank):
    n_per_tile = n_total // _NW

    @functools.partial(
        pl.kernel,
        out_type=jax.ShapeDtypeStruct((_NW, _B1), jnp.int32),
        mesh=_get_mesh(),
        compiler_params=pltpu.CompilerParams(needs_layout_passes=False),
        scratch_types=[
            pltpu.VMEM((2, _CHUNK), jnp.int32),
            pltpu.VMEM((_hwords(_B1),), jnp.int32),
            pltpu.VMEM((1, _B1), jnp.int32),
            pltpu.SemaphoreType.DMA,
            pltpu.SemaphoreType.DMA,
        ],
    )
    def hist1(cb_hbm, h1_hbm, buf, lhist, hsum, sem0, sem1):
        w = _wid()
        _zero_ref(lhist, _hwords(_B1))

        def digit(v):
            return jax.lax.shift_right_logical(v, _SH1), None

        _stream_hist(cb_hbm, buf, lhist, (sem0, sem1),
                     n_per_tile, w * n_per_tile, _B1, digit)
        _reduce_lanes(lhist, hsum, _B1)
        pltpu.sync_copy(hsum, h1_hbm.at[pl.ds(w, 1)])

    @functools.partial(
        pl.kernel,
        out_type=jax.ShapeDtypeStruct((_NW, _B2), jnp.int32),
        mesh=_get_mesh(),
        compiler_params=pltpu.CompilerParams(needs_layout_passes=False),
        scratch_types=[
            pltpu.VMEM((2, _CHUNK), jnp.int32),
            pltpu.VMEM((_hwords(_B2),), jnp.int32),
            pltpu.VMEM((1, _B2), jnp.int32),
            pltpu.VMEM((_L, _B1), jnp.int32),
            pltpu.SemaphoreType.DMA,
            pltpu.SemaphoreType.DMA,
        ],
    )
    def hist2(cb_hbm, h1_hbm, h2_hbm, buf, lhist, hsum, combuf, sem0, sem1):
        w = _wid()
        p1, _ = _combine(h1_hbm, combuf, hsum, _B1, k_rank)
        _zero_ref(lhist, _hwords(_B2))

        def digit(v):
            msk = jax.lax.shift_right_logical(v, _SH1) == p1
            return jnp.bitwise_and(jax.lax.shift_right_logical(v, _SH2), 0x7FF), msk

        _stream_hist(cb_hbm, buf, lhist, (sem0, sem1),
                     n_per_tile, w * n_per_tile, _B2, digit)
        _reduce_lanes(lhist, hsum, _B2)
        pltpu.sync_copy(hsum, h2_hbm.at[pl.ds(w, 1)])

    @functools.partial(
        pl.kernel,
        out_type=jax.ShapeDtypeStruct((_NW, _B3), jnp.int32),
        mesh=_get_mesh(),
        compiler_params=pltpu.CompilerParams(needs_layout_passes=False),
        scratch_types=[
            pltpu.VMEM((2, _CHUNK), jnp.int32),
            pltpu.VMEM((_hwords(_B3),), jnp.int32),
            pltpu.VMEM((1, _B3), jnp.int32),
            pltpu.VMEM((_L, _B1), jnp.int32),
            pltpu.VMEM((1, _B1), jnp.int32),
            pltpu.SemaphoreType.DMA,
            pltpu.SemaphoreType.DMA,
        ],
    )
    def hist3(cb_hbm, h1_hbm, h2_hbm, h3_hbm, buf, lhist, hsum3, combuf, hsum,
              sem0, sem1):
        w = _wid()
        p1, k1 = _combine(h1_hbm, combuf, hsum, _B1, k_rank)
        p2, _ = _combine(h2_hbm, combuf, hsum, _B2, k1)
        prefix = jnp.bitwise_or(jax.lax.shift_left(p1, 11), p2)
        _zero_ref(lhist, _hwords(_B3))

        def digit(v):
            msk = jax.lax.shift_right_logical(v, _SH2) == prefix
            return jnp.bitwise_and(v, 0xFF), msk

        _stream_hist(cb_hbm, buf, lhist, (sem0, sem1),
                     n_per_tile, w * n_per_tile, _B3, digit)
        _reduce_lanes(lhist, hsum3, _B3)
        pltpu.sync_copy(hsum3, h3_hbm.at[pl.ds(w, 1)])

    return hist1, hist2, hist3


def _scan_level(h_ref, rows, k, tri128, off_tri):
    hs = jnp.sum(h_ref[...].astype(jnp.float32), axis=0).reshape(rows, 128)
    cw = jax.lax.dot_general(hs, tri128, (((1,), (0,)), ((), ())),
                             preferred_element_type=jnp.float32)
    tot = cw[:, 127:128].reshape(1, rows)
    off = jax.lax.dot_general(tot, off_tri[:rows, :rows],
                              (((1,), (0,)), ((), ())),
                              preferred_element_type=jnp.float32)
    cum = cw + off.reshape(rows, 1)
    kf = k.astype(jnp.float32)
    sel = cum <= kf
    p = jnp.sum(sel.astype(jnp.int32))
    csum = jnp.max(jnp.where(sel, cum, 0.0))
    return p, k - csum.astype(jnp.int32)


def _fin_body(k_rank, h1_ref, h2_ref, h3_ref, lg_ref, tg_ref, dn_ref, dd_ref,
              out_ref, cnt_sm, bs_sm, tf_sm):
    b = pl.program_id(0)

    @pl.when(b == 0)
    def _():
        rI = lax.broadcasted_iota(jnp.int32, (128, 128), 0)
        cI = lax.broadcasted_iota(jnp.int32, (128, 128), 1)
        tri128 = (rI <= cI).astype(jnp.float32)
        off_tri = (rI < cI).astype(jnp.float32)
        k = jnp.int32(k_rank)
        p1, k1 = _scan_level(h1_ref, _B1 // 128, k, tri128, off_tri)
        p2, k2 = _scan_level(h2_ref, _B2 // 128, k1, tri128, off_tri)
        p3, _ = _scan_level(h3_ref, _B3 // 128, k2, tri128, off_tri)
        vb = jnp.bitwise_or(
            lax.shift_left(p1, _SH1),
            jnp.bitwise_or(lax.shift_left(p2, _SH2), p3))
        tf = lax.bitcast_convert_type(jnp.full((1, 1), vb, jnp.int32),
                                      jnp.float32)
        tf_sm[0] = jnp.maximum(tf, _THRESH)[0, 0]
        cnt_sm[0] = 0.0
        bs_sm[0] = 0.0

    lg = lg_ref[0]
    tg = tg_ref[0].astype(jnp.float32)
    prob = jax.nn.sigmoid(lg)
    conf = jnp.where(tg > 0.5, prob, 1.0 - prob)
    bce = jnp.maximum(lg, 0.0) - lg * tg + jnp.log1p(jnp.exp(-jnp.abs(lg)))
    m = (conf < tf_sm[0]).astype(jnp.float32)
    cnt_sm[0] += jnp.sum(m)
    bs_sm[0] += jnp.sum(bce * m)

    @pl.when(b == pl.num_programs(0) - 1)
    def _():
        num = 2.0 * jnp.sum(dn_ref[:, 0, :], axis=1, keepdims=True) + _SMOOTH
        den = jnp.sum(dd_ref[:, 0, :], axis=1, keepdims=True) + _SMOOTH
        dice = jnp.mean(1.0 - num / den)
        bce_l = bs_sm[0] / jnp.maximum(cnt_sm[0], 1.0)
        out_ref[0] = _DICE_W * dice + _BCE_W * bce_l


def _finalize(k_rank, h1, h2, h3, lg3, tg3, dn, dd):
    B, R, C = lg3.shape
    blk = lambda b: (b, 0, 0)
    full2 = lambda b: (0, 0)
    return pl.pallas_call(
        functools.partial(_fin_body, k_rank),
        grid=(B,),
        in_specs=[
            pl.BlockSpec((_NW, _B1), full2),
            pl.BlockSpec((_NW, _B2), full2),
            pl.BlockSpec((_NW, _B3), full2),
            pl.BlockSpec((1, R, C), blk),
            pl.BlockSpec((1, R, C), blk),
            pl.BlockSpec((B, 1, C), lambda b: (0, 0, 0)),
            pl.BlockSpec((B, 1, C), lambda b: (0, 0, 0)),
        ],
        out_specs=pl.BlockSpec(memory_space=pltpu.SMEM),
        out_shape=jax.ShapeDtypeStruct((1,), jnp.float32),
        scratch_shapes=[pltpu.SMEM((1,), jnp.float32),
                        pltpu.SMEM((1,), jnp.float32),
                        pltpu.SMEM((1,), jnp.float32)],
    )(h1, h2, h3, lg3, tg3, dn, dd)


def kernel(pred_logits, target):
    B, _, H, W = pred_logits.shape
    n = B * H * W
    assert n % (_NW * _CHUNK) == 0
    k_rank = min(_MIN_KEPT * B, n - 1)

    lg3 = pred_logits.reshape(B, (H * W) // 128, 128)
    tg3 = target.reshape(B, (H * W) // 128, 128)

    cb3, dn, dd = _elementwise(lg3, tg3)

    hist1, hist2, hist3 = _make_sc_passes(n, k_rank)
    cb_flat = cb3.reshape(n)
    h1 = hist1(cb_flat)
    h2 = hist2(cb_flat, h1)
    h3 = hist3(cb_flat, h1, h2)

    out = _finalize(k_rank, h1, h2, h3, lg3, tg3, dn, dd)
    return out[0]
